# baseline (speedup 1.0000x reference)
import functools

import jax
import jax.numpy as jnp
from jax import lax
from jax.experimental import pallas as pl
from jax.experimental.pallas import tpu as pltpu
from jax.experimental.pallas import tpu_sc as plsc

D_MODEL = 768
K_QUERY_ = 32
K_TOPK_ = 4
B_ = 2
T_ = 4096
BLK = 512
TOK = B_ * T_


def _fused_body(x_ref, wq_ref, bq_ref, wk_ref, bk_ref, idx_ref, k_scr):
    i = pl.program_id(0)
    dn = (((1,), (1,)), ((), ()))

    @pl.when(i == 0)
    def _():
        xb = x_ref[0].astype(jnp.bfloat16)
        kf = lax.dot_general(xb, wk_ref[...].astype(jnp.bfloat16), dn,
                             preferred_element_type=jnp.float32)
        k_scr[...] = (kf + bk_ref[...]).astype(jnp.bfloat16)

    xq = x_ref[0, pl.ds(i * BLK, BLK), :].astype(jnp.bfloat16)
    qf = lax.dot_general(xq, wq_ref[...].astype(jnp.bfloat16), dn,
                         preferred_element_type=jnp.float32)
    q = (qf + bq_ref[...]).astype(jnp.bfloat16)
    s = lax.dot_general(q, k_scr[...], dn,
                        preferred_element_type=jnp.float32)
    iota_f = lax.broadcasted_iota(jnp.int32, (BLK, T_), 1).astype(jnp.float32)
    neg_inf = jnp.float32(-jnp.inf)
    big = jnp.float32(2.0 * T_)
    idxs = []
    for r in range(K_TOPK_):
        m = jnp.max(s, axis=1, keepdims=True)
        eq = s == m
        a = jnp.min(jnp.where(eq, iota_f, big), axis=1, keepdims=True)
        idxs.append(a)
        if r < K_TOPK_ - 1:
            s = jnp.where(eq, neg_inf, s)
    idx_ref[0] = jnp.concatenate(idxs, axis=1).astype(jnp.int32)


_topk1 = pl.pallas_call(
    _fused_body,
    grid=(T_ // BLK,),
    in_specs=[
        pl.BlockSpec((1, T_, D_MODEL), lambda i: (0, 0, 0)),
        pl.BlockSpec((K_QUERY_, D_MODEL), lambda i: (0, 0)),
        pl.BlockSpec((1, K_QUERY_), lambda i: (0, 0)),
        pl.BlockSpec((K_QUERY_, D_MODEL), lambda i: (0, 0)),
        pl.BlockSpec((1, K_QUERY_), lambda i: (0, 0)),
    ],
    out_specs=pl.BlockSpec((1, BLK, K_TOPK_), lambda i: (0, i, 0)),
    out_shape=jax.ShapeDtypeStruct((1, T_, K_TOPK_), jnp.int32),
    scratch_shapes=[pltpu.VMEM((T_, K_QUERY_), jnp.bfloat16)],
)

_NC = 2
_NS = 16
_NW = _NC * _NS
CH = 16


@functools.lru_cache(maxsize=2)
def _build_gather_mean(ntok):
    TPW = ntok // _NW
    NCHUNK = TPW // CH
    mesh = plsc.VectorSubcoreMesh(core_axis_name="c", subcore_axis_name="s")

    @functools.partial(
        pl.kernel,
        mesh=mesh,
        out_type=jax.ShapeDtypeStruct((ntok, D_MODEL), jnp.float32),
        scratch_types=[
            pltpu.VMEM((TPW * K_TOPK_,), jnp.int32),
            pltpu.VMEM((2, CH * K_TOPK_, D_MODEL), jnp.float32),
            pltpu.VMEM((2, CH, D_MODEL), jnp.float32),
            pltpu.SemaphoreType.DMA,
            pltpu.SemaphoreType.DMA,
            pltpu.SemaphoreType.DMA,
            pltpu.SemaphoreType.DMA,
            pltpu.SemaphoreType.DMA,
        ],
    )
    def _gather_mean(xf_hbm, idxf_hbm, out_hbm, idx_v, rows_v, acc_v,
                     gsem0, gsem1, wsem0, wsem1, isem):
        gsems = (gsem0, gsem1)
        wsems = (wsem0, wsem1)
        wid = lax.axis_index("s") * _NC + lax.axis_index("c")
        base = wid * TPW
        pltpu.async_copy(
            idxf_hbm.at[pl.ds(base * K_TOPK_, TPW * K_TOPK_)], idx_v,
            isem).wait()

        def start_gather(ci, par):
            pltpu.async_copy(
                xf_hbm.at[idx_v.at[pl.ds(ci * (CH * K_TOPK_), CH * K_TOPK_)]],
                rows_v.at[par], gsems[par])

        def wait_gather(par):
            pltpu.make_async_copy(
                xf_hbm.at[idx_v.at[pl.ds(0, CH * K_TOPK_)]],
                rows_v.at[par], gsems[par]).wait()

        def wait_write(par):
            pltpu.make_async_copy(
                acc_v.at[par], out_hbm.at[pl.ds(0, CH)], wsems[par]).wait()

        def compute(ci, par):
            rows = rows_v.at[par]
            acc = acc_v.at[par]

            @plsc.parallel_loop(0, CH, unroll=4)
            def _(c):
                for j in range(D_MODEL // 16):
                    sl = pl.ds(j * 16, 16)
                    v = (rows[4 * c, sl] + rows[4 * c + 1, sl]) + (
                        rows[4 * c + 2, sl] + rows[4 * c + 3, sl])
                    acc[c, sl] = v * 0.25

            pltpu.async_copy(acc, out_hbm.at[pl.ds(base + ci * CH, CH)],
                             wsems[par])

        start_gather(0, 0)
        start_gather(1, 1)

        def outer(g, carry):
            for par in range(2):
                ci = 2 * g + par
                wait_gather(par)

                @pl.when(ci >= 2)
                def _():
                    wait_write(par)

                compute(ci, par)

                @pl.when(ci + 2 < NCHUNK)
                def _():
                    start_gather(ci + 2, par)
            return carry

        lax.fori_loop(0, NCHUNK // 2, outer, 0)
        for par in range(2):
            wait_write(par)

    return _gather_mean


def kernel(x, Wq, bq, Wk, bk):
    bq2 = bq.reshape(1, K_QUERY_)
    bk2 = bk.reshape(1, K_QUERY_)
    xf = x.reshape(TOK, D_MODEL)
    gather = _build_gather_mean(T_)
    idx0 = _topk1(x[0:1], Wq, bq2, Wk, bk2)
    out0 = gather(xf, idx0.reshape(T_ * K_TOPK_))
    idx1 = _topk1(x[1:2], Wq, bq2, Wk, bk2)
    out1 = gather(xf, (idx1 + T_).reshape(T_ * K_TOPK_))
    return jnp.stack([out0, out1]).reshape(B_, T_, D_MODEL)


# speedup vs baseline: 1.0321x; 1.0321x over previous
import functools

import jax
import jax.numpy as jnp
from jax import lax
from jax.experimental import pallas as pl
from jax.experimental.pallas import tpu as pltpu
from jax.experimental.pallas import tpu_sc as plsc

D_MODEL = 768
K_QUERY_ = 32
K_TOPK_ = 4
B_ = 2
T_ = 4096
BLK = 512
TOK = B_ * T_


def _proj_body(x_ref, wq_ref, bq_ref, wk_ref, bk_ref, q_ref, k_ref):
    x = x_ref[0].astype(jnp.bfloat16)
    dn = (((1,), (1,)), ((), ()))
    wq = wq_ref[...].astype(jnp.bfloat16)
    wk = wk_ref[...].astype(jnp.bfloat16)
    q = lax.dot_general(x, wq, dn, preferred_element_type=jnp.float32)
    q_ref[0] = q + bq_ref[...]
    k = lax.dot_general(x, wk, dn, preferred_element_type=jnp.float32)
    k_ref[0] = k + bk_ref[...]


def _topk_body(q_ref, k_ref, idx_ref):
    b = pl.program_id(0)
    q = q_ref[0]
    k = k_ref[0]
    dn = (((1,), (1,)), ((), ()))
    s = lax.dot_general(q.astype(jnp.bfloat16), k.astype(jnp.bfloat16), dn,
                        preferred_element_type=jnp.float32)
    iota_f = lax.broadcasted_iota(jnp.int32, (BLK, T_), 1).astype(jnp.float32)
    neg_inf = jnp.float32(-jnp.inf)
    big = jnp.float32(2.0 * T_)
    idxs = []
    for r in range(K_TOPK_):
        m = jnp.max(s, axis=1, keepdims=True)
        eq = s == m
        a = jnp.min(jnp.where(eq, iota_f, big), axis=1, keepdims=True)
        idxs.append(a)
        if r < K_TOPK_ - 1:
            s = jnp.where(eq, neg_inf, s)
    del b
    idx_ref[0] = jnp.concatenate(idxs, axis=1).astype(jnp.int32)


_proj = pl.pallas_call(
    _proj_body,
    grid=(B_,),
    in_specs=[
        pl.BlockSpec((1, T_, D_MODEL), lambda b: (b, 0, 0)),
        pl.BlockSpec((K_QUERY_, D_MODEL), lambda b: (0, 0)),
        pl.BlockSpec((1, K_QUERY_), lambda b: (0, 0)),
        pl.BlockSpec((K_QUERY_, D_MODEL), lambda b: (0, 0)),
        pl.BlockSpec((1, K_QUERY_), lambda b: (0, 0)),
    ],
    out_specs=[
        pl.BlockSpec((1, T_, K_QUERY_), lambda b: (b, 0, 0)),
        pl.BlockSpec((1, T_, K_QUERY_), lambda b: (b, 0, 0)),
    ],
    out_shape=[
        jax.ShapeDtypeStruct((B_, T_, K_QUERY_), jnp.float32),
        jax.ShapeDtypeStruct((B_, T_, K_QUERY_), jnp.float32),
    ],
)

_topk1 = pl.pallas_call(
    _topk_body,
    grid=(1, T_ // BLK),
    in_specs=[
        pl.BlockSpec((1, BLK, K_QUERY_), lambda b, i: (b, i, 0)),
        pl.BlockSpec((1, T_, K_QUERY_), lambda b, i: (b, 0, 0)),
    ],
    out_specs=pl.BlockSpec((1, BLK, K_TOPK_), lambda b, i: (b, i, 0)),
    out_shape=jax.ShapeDtypeStruct((1, T_, K_TOPK_), jnp.int32),
)

_NC = 2
_NS = 16
_NW = _NC * _NS
CH = 16


@functools.lru_cache(maxsize=2)
def _build_gather_mean(ntok):
    TPW = ntok // _NW
    NCHUNK = TPW // CH
    mesh = plsc.VectorSubcoreMesh(core_axis_name="c", subcore_axis_name="s")

    @functools.partial(
        pl.kernel,
        mesh=mesh,
        out_type=jax.ShapeDtypeStruct((ntok, D_MODEL), jnp.float32),
        scratch_types=[
            pltpu.VMEM((TPW * K_TOPK_,), jnp.int32),
            pltpu.VMEM((2, CH * K_TOPK_, D_MODEL), jnp.float32),
            pltpu.VMEM((2, CH, D_MODEL), jnp.float32),
            pltpu.SemaphoreType.DMA,
            pltpu.SemaphoreType.DMA,
            pltpu.SemaphoreType.DMA,
            pltpu.SemaphoreType.DMA,
            pltpu.SemaphoreType.DMA,
        ],
    )
    def _gather_mean(xf_hbm, idxf_hbm, out_hbm, idx_v, rows_v, acc_v,
                     gsem0, gsem1, wsem0, wsem1, isem):
        gsems = (gsem0, gsem1)
        wsems = (wsem0, wsem1)
        wid = lax.axis_index("s") * _NC + lax.axis_index("c")
        base = wid * TPW
        pltpu.async_copy(
            idxf_hbm.at[pl.ds(base * K_TOPK_, TPW * K_TOPK_)], idx_v,
            isem).wait()

        def start_gather(ci, par):
            pltpu.async_copy(
                xf_hbm.at[idx_v.at[pl.ds(ci * (CH * K_TOPK_), CH * K_TOPK_)]],
                rows_v.at[par], gsems[par])

        def wait_gather(par):
            pltpu.make_async_copy(
                xf_hbm.at[idx_v.at[pl.ds(0, CH * K_TOPK_)]],
                rows_v.at[par], gsems[par]).wait()

        def wait_write(par):
            pltpu.make_async_copy(
                acc_v.at[par], out_hbm.at[pl.ds(0, CH)], wsems[par]).wait()

        def compute(ci, par):
            rows = rows_v.at[par]
            acc = acc_v.at[par]

            @plsc.parallel_loop(0, CH, unroll=4)
            def _(c):
                for j in range(D_MODEL // 16):
                    sl = pl.ds(j * 16, 16)
                    v = (rows[4 * c, sl] + rows[4 * c + 1, sl]) + (
                        rows[4 * c + 2, sl] + rows[4 * c + 3, sl])
                    acc[c, sl] = v * 0.25

            pltpu.async_copy(acc, out_hbm.at[pl.ds(base + ci * CH, CH)],
                             wsems[par])

        start_gather(0, 0)
        start_gather(1, 1)

        def outer(g, carry):
            for par in range(2):
                ci = 2 * g + par
                wait_gather(par)

                @pl.when(ci >= 2)
                def _():
                    wait_write(par)

                compute(ci, par)

                @pl.when(ci + 2 < NCHUNK)
                def _():
                    start_gather(ci + 2, par)
            return carry

        lax.fori_loop(0, NCHUNK // 2, outer, 0)
        for par in range(2):
            wait_write(par)

    return _gather_mean


H_ = T_ // 2

_topk_h = pl.pallas_call(
    _topk_body,
    grid=(1, H_ // BLK),
    in_specs=[
        pl.BlockSpec((1, BLK, K_QUERY_), lambda b, i: (b, i, 0)),
        pl.BlockSpec((1, T_, K_QUERY_), lambda b, i: (b, 0, 0)),
    ],
    out_specs=pl.BlockSpec((1, BLK, K_TOPK_), lambda b, i: (b, i, 0)),
    out_shape=jax.ShapeDtypeStruct((1, H_, K_TOPK_), jnp.int32),
)


def kernel(x, Wq, bq, Wk, bk):
    q, k = _proj(x, Wq, bq.reshape(1, K_QUERY_), Wk, bk.reshape(1, K_QUERY_))
    xf = x.reshape(TOK, D_MODEL)
    gather_t = _build_gather_mean(T_)
    gather_h = _build_gather_mean(H_)
    idx0 = _topk1(q[0:1], k[0:1])
    out0 = gather_t(xf, idx0.reshape(T_ * K_TOPK_))
    idx1a = _topk_h(q[1:2, :H_], k[1:2])
    out1a = gather_h(xf, (idx1a + T_).reshape(H_ * K_TOPK_))
    idx1b = _topk_h(q[1:2, H_:], k[1:2])
    out1b = gather_h(xf, (idx1b + T_).reshape(H_ * K_TOPK_))
    out = jnp.concatenate([out0, out1a, out1b], axis=0)
    return out.reshape(B_, T_, D_MODEL)


# speedup vs baseline: 1.0532x; 1.0204x over previous
import functools

import jax
import jax.numpy as jnp
from jax import lax
from jax.experimental import pallas as pl
from jax.experimental.pallas import tpu as pltpu
from jax.experimental.pallas import tpu_sc as plsc

D_MODEL = 768
K_QUERY_ = 32
K_TOPK_ = 4
B_ = 2
T_ = 4096
BLK = 512
TOK = B_ * T_


def _proj_body(x_ref, wq_ref, bq_ref, wk_ref, bk_ref, q_ref, k_ref):
    x = x_ref[0].astype(jnp.bfloat16)
    dn = (((1,), (1,)), ((), ()))
    wq = wq_ref[...].astype(jnp.bfloat16)
    wk = wk_ref[...].astype(jnp.bfloat16)
    q = lax.dot_general(x, wq, dn, preferred_element_type=jnp.float32)
    q_ref[0] = q + bq_ref[...]
    k = lax.dot_general(x, wk, dn, preferred_element_type=jnp.float32)
    k_ref[0] = k + bk_ref[...]


def _topk_body(q_ref, k_ref, idx_ref):
    b = pl.program_id(0)
    q = q_ref[0]
    k = k_ref[0]
    dn = (((1,), (1,)), ((), ()))
    s = lax.dot_general(q.astype(jnp.bfloat16), k.astype(jnp.bfloat16), dn,
                        preferred_element_type=jnp.float32)
    iota_f = lax.broadcasted_iota(jnp.int32, (BLK, T_), 1).astype(jnp.float32)
    neg_inf = jnp.float32(-jnp.inf)
    big = jnp.float32(2.0 * T_)
    idxs = []
    for r in range(K_TOPK_):
        m = jnp.max(s, axis=1, keepdims=True)
        eq = s == m
        a = jnp.min(jnp.where(eq, iota_f, big), axis=1, keepdims=True)
        idxs.append(a)
        if r < K_TOPK_ - 1:
            s = jnp.where(eq, neg_inf, s)
    del b
    idx_ref[0] = jnp.concatenate(idxs, axis=1).astype(jnp.int32)


_proj = pl.pallas_call(
    _proj_body,
    grid=(B_,),
    in_specs=[
        pl.BlockSpec((1, T_, D_MODEL), lambda b: (b, 0, 0)),
        pl.BlockSpec((K_QUERY_, D_MODEL), lambda b: (0, 0)),
        pl.BlockSpec((1, K_QUERY_), lambda b: (0, 0)),
        pl.BlockSpec((K_QUERY_, D_MODEL), lambda b: (0, 0)),
        pl.BlockSpec((1, K_QUERY_), lambda b: (0, 0)),
    ],
    out_specs=[
        pl.BlockSpec((1, T_, K_QUERY_), lambda b: (b, 0, 0)),
        pl.BlockSpec((1, T_, K_QUERY_), lambda b: (b, 0, 0)),
    ],
    out_shape=[
        jax.ShapeDtypeStruct((B_, T_, K_QUERY_), jnp.float32),
        jax.ShapeDtypeStruct((B_, T_, K_QUERY_), jnp.float32),
    ],
)

_topk1 = pl.pallas_call(
    _topk_body,
    grid=(1, T_ // BLK),
    in_specs=[
        pl.BlockSpec((1, BLK, K_QUERY_), lambda b, i: (b, i, 0)),
        pl.BlockSpec((1, T_, K_QUERY_), lambda b, i: (b, 0, 0)),
    ],
    out_specs=pl.BlockSpec((1, BLK, K_TOPK_), lambda b, i: (b, i, 0)),
    out_shape=jax.ShapeDtypeStruct((1, T_, K_TOPK_), jnp.int32),
)

_NC = 2
_NS = 16
_NW = _NC * _NS
CH = 16


@functools.lru_cache(maxsize=2)
def _build_gather_mean(ntok):
    TPW = ntok // _NW
    NCHUNK = TPW // CH
    mesh = plsc.VectorSubcoreMesh(core_axis_name="c", subcore_axis_name="s")

    @functools.partial(
        pl.kernel,
        mesh=mesh,
        out_type=jax.ShapeDtypeStruct((ntok, D_MODEL), jnp.float32),
        scratch_types=[
            pltpu.VMEM((TPW * K_TOPK_,), jnp.int32),
            pltpu.VMEM((2, CH * K_TOPK_, D_MODEL), jnp.float32),
            pltpu.VMEM((2, CH, D_MODEL), jnp.float32),
            pltpu.SemaphoreType.DMA,
            pltpu.SemaphoreType.DMA,
            pltpu.SemaphoreType.DMA,
            pltpu.SemaphoreType.DMA,
            pltpu.SemaphoreType.DMA,
        ],
    )
    def _gather_mean(xf_hbm, idxf_hbm, out_hbm, idx_v, rows_v, acc_v,
                     gsem0, gsem1, wsem0, wsem1, isem):
        gsems = (gsem0, gsem1)
        wsems = (wsem0, wsem1)
        wid = lax.axis_index("s") * _NC + lax.axis_index("c")
        base = wid * TPW
        pltpu.async_copy(
            idxf_hbm.at[pl.ds(base * K_TOPK_, TPW * K_TOPK_)], idx_v,
            isem).wait()

        def start_gather(ci, par):
            pltpu.async_copy(
                xf_hbm.at[idx_v.at[pl.ds(ci * (CH * K_TOPK_), CH * K_TOPK_)]],
                rows_v.at[par], gsems[par])

        def wait_gather(par):
            pltpu.make_async_copy(
                xf_hbm.at[idx_v.at[pl.ds(0, CH * K_TOPK_)]],
                rows_v.at[par], gsems[par]).wait()

        def wait_write(par):
            pltpu.make_async_copy(
                acc_v.at[par], out_hbm.at[pl.ds(0, CH)], wsems[par]).wait()

        def compute(ci, par):
            rows = rows_v.at[par]
            acc = acc_v.at[par]

            @plsc.parallel_loop(0, CH, unroll=4)
            def _(c):
                for j in range(D_MODEL // 16):
                    sl = pl.ds(j * 16, 16)
                    v = (rows[4 * c, sl] + rows[4 * c + 1, sl]) + (
                        rows[4 * c + 2, sl] + rows[4 * c + 3, sl])
                    acc[c, sl] = v * 0.25

            pltpu.async_copy(acc, out_hbm.at[pl.ds(base + ci * CH, CH)],
                             wsems[par])

        start_gather(0, 0)
        start_gather(1, 1)

        def outer(g, carry):
            for par in range(2):
                ci = 2 * g + par
                wait_gather(par)

                @pl.when(ci >= 2)
                def _():
                    wait_write(par)

                compute(ci, par)

                @pl.when(ci + 2 < NCHUNK)
                def _():
                    start_gather(ci + 2, par)
            return carry

        lax.fori_loop(0, NCHUNK // 2, outer, 0)
        for par in range(2):
            wait_write(par)

    return _gather_mean


def kernel(x, Wq, bq, Wk, bk):
    q, k = _proj(x, Wq, bq.reshape(1, K_QUERY_), Wk, bk.reshape(1, K_QUERY_))
    xf = x.reshape(TOK, D_MODEL)
    gather = _build_gather_mean(T_)
    idx0 = _topk1(q[0:1], k[0:1])
    out0 = gather(xf, idx0.reshape(T_ * K_TOPK_))
    idx1 = _topk1(q[1:2], k[1:2])
    out1 = gather(xf, (idx1 + T_).reshape(T_ * K_TOPK_))
    return jnp.stack([out0, out1]).reshape(B_, T_, D_MODEL)
